# baseline (speedup 1.0000x reference)
import functools

import numpy as np
import jax
import jax.numpy as jnp
from jax import lax
from jax.experimental import pallas as pl
from jax.experimental.pallas import tpu as pltpu
from jax.experimental.pallas import tpu_sc as plsc

_PS, _N, _B, _C, _H, _W = 8, 256, 8, 3, 512, 512
_NT = 32
_PPT = (_B * _N) // _NT
_CPP = 48
_NDMA = (_PPT * _CPP) // 128
_V16 = (_B * _C * _H * _W) // 16
_NP = _C * _PS * _PS


def _np_threefry2x32(k1, k2, c1, c2):
    x0 = c1.astype(np.uint32)
    x1 = c2.astype(np.uint32)
    ks0 = np.uint32(k1)
    ks1 = np.uint32(k2)
    ks2 = np.uint32(ks0 ^ ks1 ^ np.uint32(0x1BD11BDA))
    ks = (ks0, ks1, ks2)
    rots = ((13, 15, 26, 6), (17, 29, 16, 24))
    x0 = x0 + ks0
    x1 = x1 + ks1
    for i in range(5):
        for r in rots[i % 2]:
            x0 = x0 + x1
            x1 = (x1 << np.uint32(r)) | (x1 >> np.uint32(32 - r))
            x1 = x1 ^ x0
        x0 = x0 + ks[(i + 1) % 3]
        x1 = x1 + ks[(i + 2) % 3] + np.uint32(i + 1)
    return x0, x1


def _np_split(key, num):
    b1, b2 = _np_threefry2x32(
        key[0], key[1], np.zeros(num, np.uint32), np.arange(num, dtype=np.uint32)
    )
    return [(b1[i], b2[i]) for i in range(num)]


def _np_random_bits(key, shape):
    size = int(np.prod(shape))
    b1, b2 = _np_threefry2x32(
        key[0], key[1], np.zeros(size, np.uint32), np.arange(size, dtype=np.uint32)
    )
    return (b1 ^ b2).reshape(shape)


def _np_randint(key, shape, minval, maxval):
    k1, k2 = _np_split(key, 2)
    hi_bits = _np_random_bits(k1, shape)
    lo_bits = _np_random_bits(k2, shape)
    span = np.uint32(maxval - minval)
    mult = np.uint32((((2 ** 16) % int(span)) ** 2) % int(span))
    off = ((hi_bits % span) * mult + (lo_bits % span)) % span
    return (np.int32(minval) + off.astype(np.int32)).astype(np.int32)


def _make_coords():
    ck = (np.uint32(0), np.uint32(1))
    k1, k2, k3, k4 = _np_split(ck, 4)
    hi = _H - _PS + 1
    return tuple(_np_randint(k, (_N, _B), 0, hi) for k in (k1, k2, k3, k4))


_GY, _GX, _TY, _TX = _make_coords()


def _gather_plan(ys, xs):
    p = np.arange(_B * _N)
    b, n = p // _N, p % _N
    y, x = ys[n, b].astype(np.int64), xs[n, b].astype(np.int64)
    k = np.arange(_CPP)
    r, which = k >> 1, k & 1
    c, dy = r >> 3, r & 7
    o = ((b[:, None] * 3 + c[None, :]) * _H + (y[:, None] + dy[None, :])) * _W + x[:, None]
    q = o >> 4
    phase = (x & 15).astype(np.int32)
    q = q + which[None, :] * (phase[:, None] > 8)
    idx = q.astype(np.int32).reshape(_NT, _NDMA, 128)
    ph = np.repeat(phase.reshape(_NT, _PPT), 16, axis=1)
    return idx, ph


_IDX_T, _PH_T = _gather_plan(_TY, _TX)


@functools.cache
def _build_sc():
    mesh = plsc.VectorSubcoreMesh(
        core_axis_name="c", subcore_axis_name="s", num_cores=2, num_subcores=16
    )
    return functools.partial(
        pl.kernel,
        out_type=jax.ShapeDtypeStruct((_NT, 2 * _PPT), jnp.float32),
        mesh=mesh,
        scratch_types=[
            pltpu.VMEM((_NDMA * 128, 16), jnp.float32),
            pltpu.VMEM((_NDMA, 128), jnp.int32),
            pltpu.VMEM((_PPT * 16,), jnp.int32),
            pltpu.VMEM((_PPT * 16,), jnp.float32),
            pltpu.VMEM((_PPT * 16,), jnp.float32),
            pltpu.VMEM((2 * _PPT,), jnp.float32),
            pltpu.SemaphoreType.DMA,
        ],
        compiler_params=pltpu.CompilerParams(
            needs_layout_passes=False, use_tc_tiling_on_sc=False
        ),
    )(_sc_stats)


def _sc_stats(tab, idx_hbm, ph_hbm, out_hbm,
              stag, idxv, phv_ref, accs, accq, outb, sem):
    wid = lax.axis_index("s") * 2 + lax.axis_index("c")
    iota = lax.broadcasted_iota(jnp.int32, (16,), 0)

    pltpu.sync_copy(idx_hbm.at[wid], idxv)
    pltpu.sync_copy(ph_hbm.at[wid], phv_ref)
    copies = [
        pltpu.async_copy(tab.at[idxv.at[j]], stag.at[pl.ds(j * 128, 128)], sem)
        for j in range(_NDMA)
    ]
    for cp in copies:
        cp.wait()

    def body(j, carry):
        phv = phv_ref[pl.ds(j * 16, 16)]
        m0 = (iota >= phv) & (iota < phv + 8)
        m1 = iota < phv - 8
        fbase = j * _CPP
        acc_s = jnp.zeros((16,), jnp.float32)
        acc_q = jnp.zeros((16,), jnp.float32)
        zero = jnp.zeros((16,), jnp.float32)
        for k in range(_CPP // 2):
            v0 = jnp.where(m0, stag[fbase + 2 * k], zero)
            v1 = jnp.where(m1, stag[fbase + 2 * k + 1], zero)
            acc_s = acc_s + v0 + v1
            acc_q = acc_q + v0 * v0 + v1 * v1
        accs[pl.ds(j * 16, 16)] = acc_s
        accq[pl.ds(j * 16, 16)] = acc_q
        return carry

    lax.fori_loop(0, _PPT, body, 0)

    base16 = jnp.left_shift(iota, 4)

    def grp_body(grp, carry):
        def red(acc):
            def red_body(i, tot):
                return tot + plsc.load_gather(acc, [base16 + (grp * 256 + i)])

            return lax.fori_loop(0, 16, red_body, jnp.zeros((16,), jnp.float32))

        ts, tq = red(accs), red(accq)
        tm = ts * (1.0 / float(_NP))
        tv = (tq - ts * tm) * (1.0 / float(_NP - 1))
        outb[pl.ds(grp * 16, 16)] = tm
        outb[pl.ds(_PPT + grp * 16, 16)] = tv
        return carry

    lax.fori_loop(0, _PPT // 16, grp_body, 0)
    pltpu.sync_copy(outb, out_hbm.at[wid])


def _win8(a):
    for k in (1, 2, 4):
        a = a + jnp.roll(a, -k, axis=0)
    for k in (1, 2, 4):
        a = a + jnp.roll(a, -k, axis=1)
    return a


def _tc_body(gen_ref, gy_ref, gx_ref, out_ref):
    c0 = gen_ref[0, 0]
    c1 = gen_ref[0, 1]
    c2 = gen_ref[0, 2]
    s = c0 + c1 + c2
    q = c0 * c0 + c1 * c1 + c2 * c2
    ws = _win8(s)
    wq = _win8(q)
    iot = jax.lax.broadcasted_iota(jnp.int32, (_N, _W), 1)
    oy = (iot == gy_ref[0, 0][:, None]).astype(jnp.bfloat16)
    ox = (iot == gx_ref[0, 0][:, None]).astype(jnp.float32)
    maps = jnp.concatenate([ws, wq], axis=1)
    hi = maps.astype(jnp.bfloat16)
    lo = (maps - hi.astype(jnp.float32)).astype(jnp.bfloat16)
    sel = (
        jax.lax.dot(oy, hi, preferred_element_type=jnp.float32)
        + jax.lax.dot(oy, lo, preferred_element_type=jnp.float32)
    )
    psum = jnp.sum(sel[:, :_W] * ox, axis=1)
    psq = jnp.sum(sel[:, _W:] * ox, axis=1)
    n = float(_NP)
    mean = psum / n
    var = (psq - psum * psum / n) / (n - 1.0)
    out_ref[0, 0] = mean
    out_ref[0, 1] = var


def _relin_body(in_ref, out_ref):
    out_ref[...] = jnp.reshape(in_ref[...], (4096, 128))


def _tc_relin(target):
    tview = target.reshape(_B * _C * _H, _W)
    return pl.pallas_call(
        _relin_body,
        grid=(12,),
        in_specs=[pl.BlockSpec((1024, _W), lambda i: (i, 0))],
        out_specs=pl.BlockSpec((4096, 128), lambda i: (i, 0)),
        out_shape=jax.ShapeDtypeStruct((49152, 128), jnp.float32),
    )(tview)


def _tc_stats(generated):
    gy = jnp.asarray(_GY.T.reshape(_B, 1, _N))
    gx = jnp.asarray(_GX.T.reshape(_B, 1, _N))
    return pl.pallas_call(
        _tc_body,
        grid=(_B,),
        in_specs=[
            pl.BlockSpec((1, _C, _H, _W), lambda b: (b, 0, 0, 0)),
            pl.BlockSpec((1, 1, _N), lambda b: (b, 0, 0)),
            pl.BlockSpec((1, 1, _N), lambda b: (b, 0, 0)),
        ],
        out_specs=pl.BlockSpec((1, 2, _N), lambda b: (b, 0, 0)),
        out_shape=jax.ShapeDtypeStruct((_B, 2, _N), jnp.float32),
    )(generated, gy, gx)


def kernel(generated, target):
    ttab = _tc_relin(target).reshape(_V16, 16)
    sc_out = _build_sc()(ttab, _IDX_T, _PH_T)
    tc_out = _tc_stats(generated)
    gm, gv = tc_out[:, 0], tc_out[:, 1]
    tm = sc_out[:, :_PPT].reshape(_B, _N)
    tv = sc_out[:, _PPT:].reshape(_B, _N)
    return jnp.mean((gm - tm) ** 2) + jnp.mean((gv - tv) ** 2)


# speedup vs baseline: 1.2035x; 1.2035x over previous
import functools

import numpy as np
import jax
import jax.numpy as jnp
from jax import lax
from jax.experimental import pallas as pl
from jax.experimental.pallas import tpu as pltpu
from jax.experimental.pallas import tpu_sc as plsc

_PS, _N, _B, _C, _H, _W = 8, 256, 8, 3, 512, 512
_NT = 32
_PPT = (_B * _N) // _NT
_CPP = 48
_NDMA = (_PPT * _CPP) // 128
_V16 = (_B * _C * _H * _W) // 16
_NP = _C * _PS * _PS


def _np_threefry2x32(k1, k2, c1, c2):
    x0 = c1.astype(np.uint32)
    x1 = c2.astype(np.uint32)
    ks0 = np.uint32(k1)
    ks1 = np.uint32(k2)
    ks2 = np.uint32(ks0 ^ ks1 ^ np.uint32(0x1BD11BDA))
    ks = (ks0, ks1, ks2)
    rots = ((13, 15, 26, 6), (17, 29, 16, 24))
    x0 = x0 + ks0
    x1 = x1 + ks1
    for i in range(5):
        for r in rots[i % 2]:
            x0 = x0 + x1
            x1 = (x1 << np.uint32(r)) | (x1 >> np.uint32(32 - r))
            x1 = x1 ^ x0
        x0 = x0 + ks[(i + 1) % 3]
        x1 = x1 + ks[(i + 2) % 3] + np.uint32(i + 1)
    return x0, x1


def _np_split(key, num):
    b1, b2 = _np_threefry2x32(
        key[0], key[1], np.zeros(num, np.uint32), np.arange(num, dtype=np.uint32)
    )
    return [(b1[i], b2[i]) for i in range(num)]


def _np_random_bits(key, shape):
    size = int(np.prod(shape))
    b1, b2 = _np_threefry2x32(
        key[0], key[1], np.zeros(size, np.uint32), np.arange(size, dtype=np.uint32)
    )
    return (b1 ^ b2).reshape(shape)


def _np_randint(key, shape, minval, maxval):
    k1, k2 = _np_split(key, 2)
    hi_bits = _np_random_bits(k1, shape)
    lo_bits = _np_random_bits(k2, shape)
    span = np.uint32(maxval - minval)
    mult = np.uint32((((2 ** 16) % int(span)) ** 2) % int(span))
    off = ((hi_bits % span) * mult + (lo_bits % span)) % span
    return (np.int32(minval) + off.astype(np.int32)).astype(np.int32)


def _make_coords():
    ck = (np.uint32(0), np.uint32(1))
    k1, k2, k3, k4 = _np_split(ck, 4)
    hi = _H - _PS + 1
    return tuple(_np_randint(k, (_N, _B), 0, hi) for k in (k1, k2, k3, k4))


_GY, _GX, _TY, _TX = _make_coords()


def _gather_plan(ys, xs):
    p = np.arange(_B * _N)
    b, n = p // _N, p % _N
    y, x = ys[n, b].astype(np.int64), xs[n, b].astype(np.int64)
    k = np.arange(_CPP)
    r, which = k >> 1, k & 1
    c, dy = r >> 3, r & 7
    o = ((b[:, None] * 3 + c[None, :]) * _H + (y[:, None] + dy[None, :])) * _W + x[:, None]
    q = o >> 4
    phase = (x & 15).astype(np.int32)
    q = q + which[None, :] * (phase[:, None] > 8)
    idx = q.astype(np.int32).reshape(_NT, _NDMA, 128)
    ph = np.repeat(phase.reshape(_NT, _PPT), 16, axis=1)
    return idx, ph


_IDX_T, _PH_T = _gather_plan(_TY, _TX)


@functools.cache
def _build_sc():
    mesh = plsc.VectorSubcoreMesh(
        core_axis_name="c", subcore_axis_name="s", num_cores=2, num_subcores=16
    )
    return functools.partial(
        pl.kernel,
        out_type=jax.ShapeDtypeStruct((_NT, 2 * _PPT), jnp.float32),
        mesh=mesh,
        scratch_types=[
            pltpu.VMEM((_NDMA * 128, 16), jnp.float32),
            pltpu.VMEM((_NDMA, 128), jnp.int32),
            pltpu.VMEM((_PPT * 16,), jnp.int32),
            pltpu.VMEM((_PPT * 16,), jnp.float32),
            pltpu.VMEM((_PPT * 16,), jnp.float32),
            pltpu.VMEM((2 * _PPT,), jnp.float32),
            pltpu.SemaphoreType.DMA,
        ],
        compiler_params=pltpu.CompilerParams(
            needs_layout_passes=False, use_tc_tiling_on_sc=False
        ),
    )(_sc_stats)


def _sc_stats(tab, idx_hbm, ph_hbm, out_hbm,
              stag, idxv, phv_ref, accs, accq, outb, sem):
    wid = lax.axis_index("s") * 2 + lax.axis_index("c")
    iota = lax.broadcasted_iota(jnp.int32, (16,), 0)

    pltpu.sync_copy(idx_hbm.at[wid], idxv)
    pltpu.sync_copy(ph_hbm.at[wid], phv_ref)
    copies = [
        pltpu.async_copy(tab.at[idxv.at[j]], stag.at[pl.ds(j * 128, 128)], sem)
        for j in range(_NDMA)
    ]
    for cp in copies:
        cp.wait()

    def body(j, carry):
        phv = phv_ref[pl.ds(j * 16, 16)]
        m0 = (iota >= phv) & (iota < phv + 8)
        m1 = iota < phv - 8
        fbase = j * _CPP
        acc_s = jnp.zeros((16,), jnp.float32)
        acc_q = jnp.zeros((16,), jnp.float32)
        zero = jnp.zeros((16,), jnp.float32)
        for k in range(_CPP // 2):
            v0 = jnp.where(m0, stag[fbase + 2 * k], zero)
            v1 = jnp.where(m1, stag[fbase + 2 * k + 1], zero)
            acc_s = acc_s + v0 + v1
            acc_q = acc_q + v0 * v0 + v1 * v1
        accs[pl.ds(j * 16, 16)] = acc_s
        accq[pl.ds(j * 16, 16)] = acc_q
        return carry

    lax.fori_loop(0, _PPT, body, 0)

    base16 = jnp.left_shift(iota, 4)

    def grp_body(grp, carry):
        def red(acc):
            def red_body(i, tot):
                return tot + plsc.load_gather(acc, [base16 + (grp * 256 + i)])

            return lax.fori_loop(0, 16, red_body, jnp.zeros((16,), jnp.float32))

        ts, tq = red(accs), red(accq)
        tm = ts * (1.0 / float(_NP))
        tv = (tq - ts * tm) * (1.0 / float(_NP - 1))
        outb[pl.ds(grp * 16, 16)] = tm
        outb[pl.ds(_PPT + grp * 16, 16)] = tv
        return carry

    lax.fori_loop(0, _PPT // 16, grp_body, 0)
    pltpu.sync_copy(outb, out_hbm.at[wid])


def _tc_body(gen_ref, gy_ref, gx_ref, out_ref):
    c0 = gen_ref[0, 0]
    c1 = gen_ref[0, 1]
    c2 = gen_ref[0, 2]
    s = c0 + c1 + c2
    q = c0 * c0 + c1 * c1 + c2 * c2
    iot = jax.lax.broadcasted_iota(jnp.int32, (_N, _W), 1)
    yv = gy_ref[0, 0][:, None]
    xv = gx_ref[0, 0][:, None]
    oy = ((iot >= yv) & (iot < yv + _PS)).astype(jnp.bfloat16)
    ox = ((iot >= xv) & (iot < xv + _PS)).astype(jnp.float32)
    maps = jnp.concatenate([s, q], axis=1)
    hi = maps.astype(jnp.bfloat16)
    lo = (maps - hi.astype(jnp.float32)).astype(jnp.bfloat16)
    sel = (
        jax.lax.dot(oy, hi, preferred_element_type=jnp.float32)
        + jax.lax.dot(oy, lo, preferred_element_type=jnp.float32)
    )
    psum = jnp.sum(sel[:, :_W] * ox, axis=1)
    psq = jnp.sum(sel[:, _W:] * ox, axis=1)
    n = float(_NP)
    mean = psum / n
    var = (psq - psum * psum / n) / (n - 1.0)
    out_ref[0, 0] = mean
    out_ref[0, 1] = var


def _relin_body(in_ref, out_ref):
    out_ref[...] = jnp.reshape(in_ref[...], (4096, 128))


def _tc_relin(target):
    tview = target.reshape(_B * _C * _H, _W)
    return pl.pallas_call(
        _relin_body,
        grid=(12,),
        in_specs=[pl.BlockSpec((1024, _W), lambda i: (i, 0))],
        out_specs=pl.BlockSpec((4096, 128), lambda i: (i, 0)),
        out_shape=jax.ShapeDtypeStruct((49152, 128), jnp.float32),
    )(tview)


def _tc_stats(generated):
    gy = jnp.asarray(_GY.T.reshape(_B, 1, _N))
    gx = jnp.asarray(_GX.T.reshape(_B, 1, _N))
    return pl.pallas_call(
        _tc_body,
        grid=(_B,),
        in_specs=[
            pl.BlockSpec((1, _C, _H, _W), lambda b: (b, 0, 0, 0)),
            pl.BlockSpec((1, 1, _N), lambda b: (b, 0, 0)),
            pl.BlockSpec((1, 1, _N), lambda b: (b, 0, 0)),
        ],
        out_specs=pl.BlockSpec((1, 2, _N), lambda b: (b, 0, 0)),
        out_shape=jax.ShapeDtypeStruct((_B, 2, _N), jnp.float32),
    )(generated, gy, gx)


def kernel(generated, target):
    ttab = _tc_relin(target).reshape(_V16, 16)
    sc_out = _build_sc()(ttab, _IDX_T, _PH_T)
    tc_out = _tc_stats(generated)
    gm, gv = tc_out[:, 0], tc_out[:, 1]
    tm = sc_out[:, :_PPT].reshape(_B, _N)
    tv = sc_out[:, _PPT:].reshape(_B, _N)
    return jnp.mean((gm - tm) ** 2) + jnp.mean((gv - tv) ** 2)


# speedup vs baseline: 1.2181x; 1.0121x over previous
import functools

import numpy as np
import jax
import jax.numpy as jnp
from jax import lax
from jax.experimental import pallas as pl
from jax.experimental.pallas import tpu as pltpu
from jax.experimental.pallas import tpu_sc as plsc

_PS, _N, _B, _C, _H, _W = 8, 256, 8, 3, 512, 512
_NT = 32
_PPT = (_B * _N) // _NT
_CPP = 48
_NDMA = (_PPT * _CPP) // 128
_V16 = (_B * _C * _H * _W) // 16
_NP = _C * _PS * _PS


def _np_threefry2x32(k1, k2, c1, c2):
    x0 = c1.astype(np.uint32)
    x1 = c2.astype(np.uint32)
    ks0 = np.uint32(k1)
    ks1 = np.uint32(k2)
    ks2 = np.uint32(ks0 ^ ks1 ^ np.uint32(0x1BD11BDA))
    ks = (ks0, ks1, ks2)
    rots = ((13, 15, 26, 6), (17, 29, 16, 24))
    x0 = x0 + ks0
    x1 = x1 + ks1
    for i in range(5):
        for r in rots[i % 2]:
            x0 = x0 + x1
            x1 = (x1 << np.uint32(r)) | (x1 >> np.uint32(32 - r))
            x1 = x1 ^ x0
        x0 = x0 + ks[(i + 1) % 3]
        x1 = x1 + ks[(i + 2) % 3] + np.uint32(i + 1)
    return x0, x1


def _np_split(key, num):
    b1, b2 = _np_threefry2x32(
        key[0], key[1], np.zeros(num, np.uint32), np.arange(num, dtype=np.uint32)
    )
    return [(b1[i], b2[i]) for i in range(num)]


def _np_random_bits(key, shape):
    size = int(np.prod(shape))
    b1, b2 = _np_threefry2x32(
        key[0], key[1], np.zeros(size, np.uint32), np.arange(size, dtype=np.uint32)
    )
    return (b1 ^ b2).reshape(shape)


def _np_randint(key, shape, minval, maxval):
    k1, k2 = _np_split(key, 2)
    hi_bits = _np_random_bits(k1, shape)
    lo_bits = _np_random_bits(k2, shape)
    span = np.uint32(maxval - minval)
    mult = np.uint32((((2 ** 16) % int(span)) ** 2) % int(span))
    off = ((hi_bits % span) * mult + (lo_bits % span)) % span
    return (np.int32(minval) + off.astype(np.int32)).astype(np.int32)


def _make_coords():
    ck = (np.uint32(0), np.uint32(1))
    k1, k2, k3, k4 = _np_split(ck, 4)
    hi = _H - _PS + 1
    return tuple(_np_randint(k, (_N, _B), 0, hi) for k in (k1, k2, k3, k4))


_GY, _GX, _TY, _TX = _make_coords()


def _gather_plan(ys, xs):
    p = np.arange(_B * _N)
    b, n = p // _N, p % _N
    y, x = ys[n, b].astype(np.int64), xs[n, b].astype(np.int64)
    k = np.arange(_CPP)
    r, which = k >> 1, k & 1
    c, dy = r >> 3, r & 7
    o = ((b[:, None] * 3 + c[None, :]) * _H + (y[:, None] + dy[None, :])) * _W + x[:, None]
    q = o >> 4
    phase = (x & 15).astype(np.int32)
    q = q + which[None, :] * (phase[:, None] > 8)
    idx = q.astype(np.int32).reshape(_NT, _NDMA, 128)
    ph = np.repeat(phase.reshape(_NT, _PPT), 16, axis=1)
    return idx, ph


_IDX_T, _PH_T = _gather_plan(_TY, _TX)


@functools.cache
def _build_sc():
    mesh = plsc.VectorSubcoreMesh(
        core_axis_name="c", subcore_axis_name="s", num_cores=2, num_subcores=16
    )
    return functools.partial(
        pl.kernel,
        out_type=jax.ShapeDtypeStruct((_NT, 2 * _PPT), jnp.float32),
        mesh=mesh,
        scratch_types=[
            pltpu.VMEM((_NDMA * 128, 16), jnp.float32),
            pltpu.VMEM((_NDMA, 128), jnp.int32),
            pltpu.VMEM((_PPT * 16,), jnp.int32),
            pltpu.VMEM((_PPT * 16,), jnp.float32),
            pltpu.VMEM((_PPT * 16,), jnp.float32),
            pltpu.VMEM((2 * _PPT,), jnp.float32),
            pltpu.SemaphoreType.DMA,
        ],
        compiler_params=pltpu.CompilerParams(
            needs_layout_passes=False, use_tc_tiling_on_sc=False
        ),
    )(_sc_stats)


def _sc_stats(tab, idx_hbm, ph_hbm, out_hbm,
              stag, idxv, phv_ref, accs, accq, outb, sem):
    wid = lax.axis_index("s") * 2 + lax.axis_index("c")
    iota = lax.broadcasted_iota(jnp.int32, (16,), 0)

    pltpu.sync_copy(idx_hbm.at[wid], idxv)
    pltpu.sync_copy(ph_hbm.at[wid], phv_ref)
    copies = [
        pltpu.async_copy(tab.at[idxv.at[j]], stag.at[pl.ds(j * 128, 128)], sem)
        for j in range(_NDMA)
    ]
    for cp in copies:
        cp.wait()

    def body(j, carry):
        phv = phv_ref[pl.ds(j * 16, 16)]
        m0 = (iota >= phv) & (iota < phv + 8)
        m1 = iota < phv - 8
        fbase = j * _CPP
        acc_s = jnp.zeros((16,), jnp.float32)
        acc_q = jnp.zeros((16,), jnp.float32)
        zero = jnp.zeros((16,), jnp.float32)
        for k in range(_CPP // 2):
            v0 = jnp.where(m0, stag[fbase + 2 * k], zero)
            v1 = jnp.where(m1, stag[fbase + 2 * k + 1], zero)
            acc_s = acc_s + v0 + v1
            acc_q = acc_q + v0 * v0 + v1 * v1
        accs[pl.ds(j * 16, 16)] = acc_s
        accq[pl.ds(j * 16, 16)] = acc_q
        return carry

    lax.fori_loop(0, _PPT, body, 0)

    base16 = jnp.left_shift(iota, 4)

    def grp_body(grp, carry):
        def red(acc):
            def red_body(i, tot):
                return tot + plsc.load_gather(acc, [base16 + (grp * 256 + i)])

            return lax.fori_loop(0, 16, red_body, jnp.zeros((16,), jnp.float32))

        ts, tq = red(accs), red(accq)
        tm = ts * (1.0 / float(_NP))
        tv = (tq - ts * tm) * (1.0 / float(_NP - 1))
        outb[pl.ds(grp * 16, 16)] = tm
        outb[pl.ds(_PPT + grp * 16, 16)] = tv
        return carry

    lax.fori_loop(0, _PPT // 16, grp_body, 0)
    pltpu.sync_copy(outb, out_hbm.at[wid])


def _tc_body(gen_ref, gy_ref, gx_ref, out_ref):
    c0 = gen_ref[0, 0]
    c1 = gen_ref[0, 1]
    c2 = gen_ref[0, 2]
    s = c0 + c1 + c2
    q = c0 * c0 + c1 * c1 + c2 * c2
    iot = jax.lax.broadcasted_iota(jnp.int32, (_N, _W), 1)
    yv = gy_ref[0, 0][:, None]
    xv = gx_ref[0, 0][:, None]
    oy = ((iot >= yv) & (iot < yv + _PS)).astype(jnp.bfloat16)
    ox = ((iot >= xv) & (iot < xv + _PS)).astype(jnp.float32)
    def band_sel(m):
        hi = m.astype(jnp.bfloat16)
        lo = (m - hi.astype(jnp.float32)).astype(jnp.bfloat16)
        return (
            jax.lax.dot(oy, hi, preferred_element_type=jnp.float32)
            + jax.lax.dot(oy, lo, preferred_element_type=jnp.float32)
        )

    psum = jnp.sum(band_sel(s) * ox, axis=1)
    psq = jnp.sum(band_sel(q) * ox, axis=1)
    n = float(_NP)
    mean = psum / n
    var = (psq - psum * psum / n) / (n - 1.0)
    out_ref[0, 0] = mean
    out_ref[0, 1] = var


def _relin_body(in_ref, out_ref):
    out_ref[...] = jnp.reshape(in_ref[...], (4096, 128))


def _tc_relin(target):
    tview = target.reshape(_B * _C * _H, _W)
    return pl.pallas_call(
        _relin_body,
        grid=(12,),
        in_specs=[pl.BlockSpec((1024, _W), lambda i: (i, 0))],
        out_specs=pl.BlockSpec((4096, 128), lambda i: (i, 0)),
        out_shape=jax.ShapeDtypeStruct((49152, 128), jnp.float32),
    )(tview)


def _tc_stats(generated):
    gy = jnp.asarray(_GY.T.reshape(_B, 1, _N))
    gx = jnp.asarray(_GX.T.reshape(_B, 1, _N))
    return pl.pallas_call(
        _tc_body,
        grid=(_B,),
        in_specs=[
            pl.BlockSpec((1, _C, _H, _W), lambda b: (b, 0, 0, 0)),
            pl.BlockSpec((1, 1, _N), lambda b: (b, 0, 0)),
            pl.BlockSpec((1, 1, _N), lambda b: (b, 0, 0)),
        ],
        out_specs=pl.BlockSpec((1, 2, _N), lambda b: (b, 0, 0)),
        out_shape=jax.ShapeDtypeStruct((_B, 2, _N), jnp.float32),
    )(generated, gy, gx)


def kernel(generated, target):
    ttab = _tc_relin(target).reshape(_V16, 16)
    sc_out = _build_sc()(ttab, _IDX_T, _PH_T)
    tc_out = _tc_stats(generated)
    gm, gv = tc_out[:, 0], tc_out[:, 1]
    tm = sc_out[:, :_PPT].reshape(_B, _N)
    tv = sc_out[:, _PPT:].reshape(_B, _N)
    return jnp.mean((gm - tm) ** 2) + jnp.mean((gv - tv) ** 2)
